# baseline (speedup 1.0000x reference)
import functools

import jax
import jax.numpy as jnp
from jax import lax
from jax.experimental import pallas as pl
from jax.experimental.pallas import tpu as pltpu
from jax.experimental.pallas import tpu_sc as plsc

NUM_CLS = 32
D_MODEL = 256
BATCH = 4096
HIST = 50

NC, NS = 2, 16
NW = NC * NS
ROWS = BATCH * HIST
R_PER_W = ROWS // NW
K_REP = 8
CHUNK = 128
NCHUNK = R_PER_W // CHUNK
PAIRS = NCHUNK // 2

FB = 1024
NFB = BATCH // FB
NBLK = HIST * NFB


@functools.partial(
    pl.kernel,
    out_type=jax.ShapeDtypeStruct((ROWS, D_MODEL), jnp.float32),
    mesh=plsc.VectorSubcoreMesh(core_axis_name="c", subcore_axis_name="s"),
    scratch_types=[
        pltpu.VMEM((NCHUNK, CHUNK), jnp.int32),
        pltpu.VMEM((CHUNK, D_MODEL), jnp.float32),
        pltpu.VMEM((CHUNK, D_MODEL), jnp.float32),
        pltpu.VMEM((CHUNK, D_MODEL), jnp.float32),
        pltpu.SemaphoreType.DMA,
        pltpu.SemaphoreType.DMA,
        pltpu.SemaphoreType.DMA,
        pltpu.SemaphoreType.DMA,
        pltpu.SemaphoreType.DMA,
        pltpu.SemaphoreType.DMA,
    ],
)
def _embed_sc(table_hbm, idx_hbm, out_hbm, idx_v, buf_0, buf_1, buf_2,
              gs_0, gs_1, gs_2, ws_0, ws_1, ws_2):
    wid = lax.axis_index("s") * NC + lax.axis_index("c")
    base = wid * R_PER_W
    bufs = (buf_0, buf_1, buf_2)
    gsems = (gs_0, gs_1, gs_2)
    wsems = (ws_0, ws_1, ws_2)

    pltpu.sync_copy(idx_hbm.at[wid], idx_v)

    def gather(c, par):
        return pltpu.make_async_copy(
            table_hbm.at[idx_v.at[c]], bufs[par], gsems[par])

    def write(c, par):
        return pltpu.make_async_copy(
            bufs[par], out_hbm.at[pl.ds(base + c * CHUNK, CHUNK)], wsems[par])

    gather(0, 0).start()
    gather(1, 1).start()

    def step(c):
        for par in range(3):
            @pl.when(lax.rem(c, 3) == par)
            def _(par=par):
                @pl.when(c + 2 < NCHUNK)
                def _():
                    @pl.when(c >= 1)
                    def _():
                        write(c - 1, (par + 2) % 3).wait()
                    gather(c + 2, (par + 2) % 3).start()
                gather(c, par).wait()
                write(c, par).start()

    pl.loop(0, NCHUNK)(step)
    write(NCHUNK - 3, (NCHUNK - 3) % 3).wait()
    write(NCHUNK - 2, (NCHUNK - 2) % 3).wait()
    write(NCHUNK - 1, (NCHUNK - 1) % 3).wait()


NLEAD = 2
NBUF = NLEAD + 1


def _finish_body(rows_hbm, out_hbm, s_0, s_1, s_2,
                 ls_0, ls_1, ls_2, ss_0, ss_1, ss_2):
    b = pl.program_id(0)
    scrs = (s_0, s_1, s_2)
    lsems = (ls_0, ls_1, ls_2)
    ssems = (ss_0, ss_1, ss_2)

    def at(blk):
        return (pl.ds(blk // NFB, 1), pl.ds((blk % NFB) * FB, FB))

    def load(blk, par):
        return pltpu.make_async_copy(rows_hbm.at[at(blk)], scrs[par], lsems[par])

    def store(blk, par):
        return pltpu.make_async_copy(scrs[par], out_hbm.at[at(blk)], ssems[par])

    @pl.when(b == 0)
    def _():
        load(0, 0).start()
        load(1, 1).start()

    for par in range(3):
        @pl.when(lax.rem(b, 3) == par)
        def _(par=par):
            @pl.when(b + 2 < NBLK)
            def _():
                @pl.when(b >= 1)
                def _():
                    store(b - 1, (par + 2) % 3).wait()
                load(b + 2, (par + 2) % 3).start()
            load(b, par).wait()
            store(b, par).start()
            @pl.when(b == NBLK - 1)
            def _():
                store(NBLK - 3, (NBLK - 3) % 3).wait()
                store(NBLK - 2, (NBLK - 2) % 3).wait()
                store(NBLK - 1, (NBLK - 1) % 3).wait()


_finish = pl.pallas_call(
    _finish_body,
    grid=(NBLK,),
    in_specs=[pl.BlockSpec(memory_space=pl.ANY)],
    out_specs=pl.BlockSpec(memory_space=pl.ANY),
    out_shape=jax.ShapeDtypeStruct((HIST, BATCH, D_MODEL), jnp.float32),
    scratch_shapes=[
        pltpu.VMEM((1, FB, D_MODEL), jnp.float32),
        pltpu.VMEM((1, FB, D_MODEL), jnp.float32),
        pltpu.VMEM((1, FB, D_MODEL), jnp.float32),
        pltpu.SemaphoreType.DMA,
        pltpu.SemaphoreType.DMA,
        pltpu.SemaphoreType.DMA,
        pltpu.SemaphoreType.DMA,
        pltpu.SemaphoreType.DMA,
        pltpu.SemaphoreType.DMA,
    ],
)


def kernel(data, table):
    flat = data.T.reshape(-1)
    i = jnp.arange(ROWS, dtype=jnp.int32)
    offs = (i // R_PER_W) * K_REP + (i % K_REP)
    idx = (flat + NUM_CLS * offs).reshape(NW, NCHUNK, CHUNK)
    rep = jnp.tile(table, (NW * K_REP, 1))
    rows = _embed_sc(rep, idx).reshape(HIST, BATCH, D_MODEL)
    return _finish(rows).transpose(1, 0, 2)


# speedup vs baseline: 1.2629x; 1.2629x over previous
import functools

import jax
import jax.numpy as jnp
from jax import lax
from jax.experimental import pallas as pl
from jax.experimental.pallas import tpu as pltpu
from jax.experimental.pallas import tpu_sc as plsc

NUM_CLS = 32
D_MODEL = 256
BATCH = 4096
HIST = 50

NC, NS = 2, 16
NW = NC * NS
ROWS = BATCH * HIST
R_PER_W = ROWS // NW
K_REP = 8
CHUNK = 128
NCHUNK = R_PER_W // CHUNK
PAIRS = NCHUNK // 2

FB = 2048
NFB = BATCH // FB
NBLK = HIST * NFB


@functools.partial(
    pl.kernel,
    out_type=jax.ShapeDtypeStruct((ROWS, D_MODEL), jnp.float32),
    mesh=plsc.VectorSubcoreMesh(core_axis_name="c", subcore_axis_name="s"),
    scratch_types=[
        pltpu.VMEM((NCHUNK, CHUNK), jnp.int32),
        pltpu.VMEM((CHUNK, D_MODEL), jnp.float32),
        pltpu.VMEM((CHUNK, D_MODEL), jnp.float32),
        pltpu.VMEM((CHUNK, D_MODEL), jnp.float32),
        pltpu.SemaphoreType.DMA,
        pltpu.SemaphoreType.DMA,
        pltpu.SemaphoreType.DMA,
        pltpu.SemaphoreType.DMA,
        pltpu.SemaphoreType.DMA,
        pltpu.SemaphoreType.DMA,
    ],
)
def _embed_sc(table_hbm, idx_hbm, out_hbm, idx_v, buf_0, buf_1, buf_2,
              gs_0, gs_1, gs_2, ws_0, ws_1, ws_2):
    wid = lax.axis_index("s") * NC + lax.axis_index("c")
    base = wid * R_PER_W
    bufs = (buf_0, buf_1, buf_2)
    gsems = (gs_0, gs_1, gs_2)
    wsems = (ws_0, ws_1, ws_2)

    pltpu.sync_copy(idx_hbm.at[wid], idx_v)

    def gather(c, par):
        return pltpu.make_async_copy(
            table_hbm.at[idx_v.at[c]], bufs[par], gsems[par])

    def write(c, par):
        return pltpu.make_async_copy(
            bufs[par], out_hbm.at[pl.ds(base + c * CHUNK, CHUNK)], wsems[par])

    gather(0, 0).start()
    gather(1, 1).start()

    def step(c):
        for par in range(3):
            @pl.when(lax.rem(c, 3) == par)
            def _(par=par):
                @pl.when(c + 2 < NCHUNK)
                def _():
                    @pl.when(c >= 1)
                    def _():
                        write(c - 1, (par + 2) % 3).wait()
                    gather(c + 2, (par + 2) % 3).start()
                gather(c, par).wait()
                write(c, par).start()

    pl.loop(0, NCHUNK)(step)
    write(NCHUNK - 3, (NCHUNK - 3) % 3).wait()
    write(NCHUNK - 2, (NCHUNK - 2) % 3).wait()
    write(NCHUNK - 1, (NCHUNK - 1) % 3).wait()


NLEAD = 2
NBUF = NLEAD + 1


def _finish_body(rows_hbm, out_ref, in_a, in_b, in_c, sem_a, sem_b, sem_c):
    b = pl.program_id(0)
    ins = (in_a, in_b, in_c)
    sems = (sem_a, sem_b, sem_c)

    def load(blk, par):
        return pltpu.make_async_copy(
            rows_hbm.at[pl.ds(blk * FB, FB)], ins[par], sems[par])

    @pl.when(b == 0)
    def _():
        for k in range(NLEAD):
            load(k, k).start()

    @pl.when(b + NLEAD < NBLK)
    def _():
        for par in range(NBUF):
            @pl.when(lax.rem(b + NLEAD, NBUF) == par)
            def _(par=par):
                load(b + NLEAD, par).start()

    for par in range(NBUF):
        @pl.when(lax.rem(b, NBUF) == par)
        def _(par=par):
            load(b, par).wait()
            out_ref[...] = ins[par][...].reshape(1, FB, D_MODEL)


_finish = pl.pallas_call(
    _finish_body,
    grid=(NBLK,),
    in_specs=[pl.BlockSpec(memory_space=pl.ANY)],
    out_specs=pl.BlockSpec((1, FB, D_MODEL), lambda b: (b // NFB, b % NFB, 0)),
    out_shape=jax.ShapeDtypeStruct((HIST, BATCH, D_MODEL), jnp.float32),
    scratch_shapes=[
        pltpu.VMEM((FB, D_MODEL), jnp.float32),
        pltpu.VMEM((FB, D_MODEL), jnp.float32),
        pltpu.VMEM((FB, D_MODEL), jnp.float32),
        pltpu.SemaphoreType.DMA,
        pltpu.SemaphoreType.DMA,
        pltpu.SemaphoreType.DMA,
    ],
)


def kernel(data, table):
    flat = data.T.reshape(-1)
    i = jnp.arange(ROWS, dtype=jnp.int32)
    offs = (i // R_PER_W) * K_REP + (i % K_REP)
    idx = (flat + NUM_CLS * offs).reshape(NW, NCHUNK, CHUNK)
    rep = jnp.tile(table, (NW * K_REP, 1))
    rows = _embed_sc(rep, idx)
    return _finish(rows).transpose(1, 0, 2)


# speedup vs baseline: 1.2709x; 1.0063x over previous
import functools

import jax
import jax.numpy as jnp
from jax import lax
from jax.experimental import pallas as pl
from jax.experimental.pallas import tpu as pltpu
from jax.experimental.pallas import tpu_sc as plsc

NUM_CLS = 32
D_MODEL = 256
BATCH = 4096
HIST = 50

NC, NS = 2, 16
NW = NC * NS
ROWS = BATCH * HIST
R_PER_W = ROWS // NW
K_REP = 8
CHUNK = 128
NCHUNK = R_PER_W // CHUNK
PAIRS = NCHUNK // 2

FB = 4096
NFB = BATCH // FB
NBLK = HIST * NFB


@functools.partial(
    pl.kernel,
    out_type=jax.ShapeDtypeStruct((ROWS, D_MODEL), jnp.float32),
    mesh=plsc.VectorSubcoreMesh(core_axis_name="c", subcore_axis_name="s"),
    scratch_types=[
        pltpu.VMEM((NCHUNK, CHUNK), jnp.int32),
        pltpu.VMEM((CHUNK, D_MODEL), jnp.float32),
        pltpu.VMEM((CHUNK, D_MODEL), jnp.float32),
        pltpu.VMEM((CHUNK, D_MODEL), jnp.float32),
        pltpu.SemaphoreType.DMA,
        pltpu.SemaphoreType.DMA,
        pltpu.SemaphoreType.DMA,
        pltpu.SemaphoreType.DMA,
        pltpu.SemaphoreType.DMA,
        pltpu.SemaphoreType.DMA,
    ],
)
def _embed_sc(table_hbm, idx_hbm, out_hbm, idx_v, buf_0, buf_1, buf_2,
              gs_0, gs_1, gs_2, ws_0, ws_1, ws_2):
    wid = lax.axis_index("s") * NC + lax.axis_index("c")
    base = wid * R_PER_W
    bufs = (buf_0, buf_1, buf_2)
    gsems = (gs_0, gs_1, gs_2)
    wsems = (ws_0, ws_1, ws_2)

    pltpu.sync_copy(idx_hbm.at[wid], idx_v)

    def gather(c, par):
        return pltpu.make_async_copy(
            table_hbm.at[idx_v.at[c]], bufs[par], gsems[par])

    def write(c, par):
        return pltpu.make_async_copy(
            bufs[par], out_hbm.at[pl.ds(base + c * CHUNK, CHUNK)], wsems[par])

    gather(0, 0).start()
    gather(1, 1).start()

    def step(c):
        for par in range(3):
            @pl.when(lax.rem(c, 3) == par)
            def _(par=par):
                @pl.when(c + 2 < NCHUNK)
                def _():
                    @pl.when(c >= 1)
                    def _():
                        write(c - 1, (par + 2) % 3).wait()
                    gather(c + 2, (par + 2) % 3).start()
                gather(c, par).wait()
                write(c, par).start()

    pl.loop(0, NCHUNK)(step)
    write(NCHUNK - 3, (NCHUNK - 3) % 3).wait()
    write(NCHUNK - 2, (NCHUNK - 2) % 3).wait()
    write(NCHUNK - 1, (NCHUNK - 1) % 3).wait()


NLEAD = 2
NBUF = NLEAD + 1


def _finish_body(rows_hbm, out_ref, in_a, in_b, in_c, sem_a, sem_b, sem_c):
    b = pl.program_id(0)
    ins = (in_a, in_b, in_c)
    sems = (sem_a, sem_b, sem_c)

    def load(blk, par):
        return pltpu.make_async_copy(
            rows_hbm.at[pl.ds(blk * FB, FB)], ins[par], sems[par])

    @pl.when(b == 0)
    def _():
        for k in range(NLEAD):
            load(k, k).start()

    @pl.when(b + NLEAD < NBLK)
    def _():
        for par in range(NBUF):
            @pl.when(lax.rem(b + NLEAD, NBUF) == par)
            def _(par=par):
                load(b + NLEAD, par).start()

    for par in range(NBUF):
        @pl.when(lax.rem(b, NBUF) == par)
        def _(par=par):
            load(b, par).wait()
            out_ref[...] = ins[par][...].reshape(1, FB, D_MODEL)


_finish = pl.pallas_call(
    _finish_body,
    grid=(NBLK,),
    in_specs=[pl.BlockSpec(memory_space=pl.ANY)],
    out_specs=pl.BlockSpec((1, FB, D_MODEL), lambda b: (b // NFB, b % NFB, 0)),
    out_shape=jax.ShapeDtypeStruct((HIST, BATCH, D_MODEL), jnp.float32),
    scratch_shapes=[
        pltpu.VMEM((FB, D_MODEL), jnp.float32),
        pltpu.VMEM((FB, D_MODEL), jnp.float32),
        pltpu.VMEM((FB, D_MODEL), jnp.float32),
        pltpu.SemaphoreType.DMA,
        pltpu.SemaphoreType.DMA,
        pltpu.SemaphoreType.DMA,
    ],
)


def kernel(data, table):
    flat = data.T.reshape(-1)
    i = jnp.arange(ROWS, dtype=jnp.int32)
    offs = (i // R_PER_W) * K_REP + (i % K_REP)
    idx = (flat + NUM_CLS * offs).reshape(NW, NCHUNK, CHUNK)
    rep = jnp.tile(table, (NW * K_REP, 1))
    rows = _embed_sc(rep, idx)
    return _finish(rows).transpose(1, 0, 2)


# speedup vs baseline: 1.2726x; 1.0014x over previous
import functools

import jax
import jax.numpy as jnp
from jax import lax
from jax.experimental import pallas as pl
from jax.experimental.pallas import tpu as pltpu
from jax.experimental.pallas import tpu_sc as plsc

NUM_CLS = 32
D_MODEL = 256
BATCH = 4096
HIST = 50

NC, NS = 2, 16
NW = NC * NS
ROWS = BATCH * HIST
R_PER_W = ROWS // NW
K_REP = 8
CHUNK = 128
NCHUNK = R_PER_W // CHUNK

FB = 4096
NFB = BATCH // FB
NBLK = HIST * NFB


@functools.partial(
    pl.kernel,
    out_type=jax.ShapeDtypeStruct((ROWS, D_MODEL), jnp.float32),
    mesh=plsc.VectorSubcoreMesh(core_axis_name="c", subcore_axis_name="s"),
    scratch_types=[
        pltpu.VMEM((NCHUNK, CHUNK), jnp.int32),
        pltpu.VMEM((CHUNK, D_MODEL), jnp.float32),
        pltpu.VMEM((CHUNK, D_MODEL), jnp.float32),
        pltpu.VMEM((CHUNK, D_MODEL), jnp.float32),
        pltpu.SemaphoreType.DMA,
        pltpu.SemaphoreType.DMA,
        pltpu.SemaphoreType.DMA,
        pltpu.SemaphoreType.DMA,
        pltpu.SemaphoreType.DMA,
        pltpu.SemaphoreType.DMA,
    ],
)
def _embed_sc(table_hbm, idx_hbm, out_hbm, idx_v, buf_0, buf_1, buf_2,
              gs_0, gs_1, gs_2, ws_0, ws_1, ws_2):
    wid = lax.axis_index("s") * NC + lax.axis_index("c")
    base = wid * R_PER_W
    bufs = (buf_0, buf_1, buf_2)
    gsems = (gs_0, gs_1, gs_2)
    wsems = (ws_0, ws_1, ws_2)

    pltpu.sync_copy(idx_hbm.at[wid], idx_v)

    def gather(c, par):
        return pltpu.make_async_copy(
            table_hbm.at[idx_v.at[c]], bufs[par], gsems[par])

    def write(c, par):
        return pltpu.make_async_copy(
            bufs[par], out_hbm.at[pl.ds(base + c * CHUNK, CHUNK)], wsems[par])

    gather(0, 0).start()
    gather(1, 1).start()

    def step(c):
        for par in range(3):
            @pl.when(lax.rem(c, 3) == par)
            def _(par=par):
                @pl.when(c + 2 < NCHUNK)
                def _():
                    @pl.when(c >= 1)
                    def _():
                        write(c - 1, (par + 2) % 3).wait()
                    gather(c + 2, (par + 2) % 3).start()
                gather(c, par).wait()
                write(c, par).start()

    pl.loop(0, NCHUNK)(step)
    write(NCHUNK - 3, (NCHUNK - 3) % 3).wait()
    write(NCHUNK - 2, (NCHUNK - 2) % 3).wait()
    write(NCHUNK - 1, (NCHUNK - 1) % 3).wait()


NLEAD = 2
NBUF = NLEAD + 1


def _finish_body(rows_hbm, out_ref, in_a, in_b, in_c, sem_a, sem_b, sem_c):
    b = pl.program_id(0)
    ins = (in_a, in_b, in_c)
    sems = (sem_a, sem_b, sem_c)

    def load(blk, par):
        return pltpu.make_async_copy(
            rows_hbm.at[pl.ds(blk * FB, FB)], ins[par], sems[par])

    @pl.when(b == 0)
    def _():
        for k in range(NLEAD):
            load(k, k).start()

    @pl.when(b + NLEAD < NBLK)
    def _():
        for par in range(NBUF):
            @pl.when(lax.rem(b + NLEAD, NBUF) == par)
            def _(par=par):
                load(b + NLEAD, par).start()

    for par in range(NBUF):
        @pl.when(lax.rem(b, NBUF) == par)
        def _(par=par):
            load(b, par).wait()
            out_ref[...] = ins[par][...].reshape(1, FB, D_MODEL)


_finish = pl.pallas_call(
    _finish_body,
    grid=(NBLK,),
    in_specs=[pl.BlockSpec(memory_space=pl.ANY)],
    out_specs=pl.BlockSpec((1, FB, D_MODEL), lambda b: (b // NFB, b % NFB, 0)),
    out_shape=jax.ShapeDtypeStruct((HIST, BATCH, D_MODEL), jnp.float32),
    scratch_shapes=[
        pltpu.VMEM((FB, D_MODEL), jnp.float32),
        pltpu.VMEM((FB, D_MODEL), jnp.float32),
        pltpu.VMEM((FB, D_MODEL), jnp.float32),
        pltpu.SemaphoreType.DMA,
        pltpu.SemaphoreType.DMA,
        pltpu.SemaphoreType.DMA,
    ],
)


def kernel(data, table):
    flat = data.T.reshape(-1)
    i = jnp.arange(ROWS, dtype=jnp.int32)
    offs = (i // R_PER_W) * K_REP + (i % K_REP)
    idx = (flat + NUM_CLS * offs).reshape(NW, NCHUNK, CHUNK)
    rep = jnp.tile(table, (NW * K_REP, 1))
    rows = _embed_sc(rep, idx)
    return _finish(rows).transpose(1, 0, 2)
